# Initial kernel scaffold; baseline (speedup 1.0000x reference)
import jax
import jax.numpy as jnp
from jax.experimental import pallas as pl
from jax.experimental.pallas import tpu as pltpu
from jax.experimental.pallas import tpu_sc as plsc

_NEG = 0.01
_N = 10000
_K = 16
_QB = 256
_NPAD = 10240
_RB = 1000
_GW = 128


def _leaky(v):
    return jnp.where(v >= 0, v, _NEG * v)


def _mlp_proj_body(x_ref, w1, b1, w2, b2, w3, b3, w4, b4, wst, bs, wht, bh,
                   h1_ref, h_ref, s_ref):
    h = _leaky(jnp.dot(x_ref[...], w1[...],
                       preferred_element_type=jnp.float32) + b1[...])
    h = _leaky(jnp.dot(h, w2[...], preferred_element_type=jnp.float32) + b2[...])
    h = _leaky(jnp.dot(h, w3[...], preferred_element_type=jnp.float32) + b3[...])
    h = _leaky(jnp.dot(h, w4[...], preferred_element_type=jnp.float32) + b4[...])
    h1_ref[...] = h
    s_ref[...] = jnp.dot(h, wst[...], preferred_element_type=jnp.float32) + bs[...]
    h_ref[...] = jnp.dot(h, wht[...], preferred_element_type=jnp.float32) + bh[...]


def _run_mlp_proj(x, nn1, Ws, bs, Wh, bh):
    n = x.shape[0]
    grid = n // _RB
    full = lambda a: pl.BlockSpec(a.shape, lambda i: (0,) * a.ndim)
    row = lambda d: pl.BlockSpec((_RB, d), lambda i: (i, 0))
    wargs = []
    in_specs = [row(x.shape[1])]
    for (W, b) in nn1:
        wargs += [W.T, b.reshape(1, -1)]
    wargs += [Ws.T, bs.reshape(1, -1), Wh.T, bh.reshape(1, -1)]
    in_specs += [full(a) for a in wargs]
    return pl.pallas_call(
        _mlp_proj_body,
        grid=(grid,),
        in_specs=in_specs,
        out_specs=[row(32), row(32), row(2)],
        out_shape=[jax.ShapeDtypeStruct((n, 32), jnp.float32),
                   jax.ShapeDtypeStruct((n, 32), jnp.float32),
                   jax.ShapeDtypeStruct((n, 2), jnp.float32)],
    )(x, *wargs)


def _knn_body(sq_ref, st_ref, nbr_ref, w_ref, d2_ref):
    s0q = sq_ref[:, 0:1]
    s1q = sq_ref[:, 1:2]
    s0a = st_ref[0:1, :]
    s1a = st_ref[1:2, :]
    d0 = s0q - s0a
    d1 = s1q - s1a
    d2_ref[...] = d0 * d0 + d1 * d1
    lane = jax.lax.broadcasted_iota(jnp.int32, (_QB, _NPAD), 1)
    for t in range(_K):
        d2 = d2_ref[...]
        m = jnp.min(d2, axis=1)
        am = jnp.argmin(d2, axis=1)
        nbr_ref[:, t] = am
        w_ref[:, t] = jnp.exp(-10.0 * m)
        d2_ref[...] = jnp.where(lane == am[:, None], jnp.inf, d2)


def _run_knn(s, st):
    grid = _NPAD // _QB
    return pl.pallas_call(
        _knn_body,
        grid=(grid,),
        in_specs=[pl.BlockSpec((_QB, 2), lambda i: (i, 0)),
                  pl.BlockSpec(st.shape, lambda i: (0, 0))],
        out_specs=[pl.BlockSpec((_QB, _K), lambda i: (i, 0)),
                   pl.BlockSpec((_QB, _K), lambda i: (i, 0))],
        out_shape=[jax.ShapeDtypeStruct((_N, _K), jnp.int32),
                   jax.ShapeDtypeStruct((_N, _K), jnp.float32)],
        scratch_shapes=[pltpu.VMEM((_QB, _NPAD), jnp.float32)],
    )(s, st)


def _sc_gather(h, idx_flat):
    n_idx = idx_flat.shape[1]
    dim = h.shape[1]
    mesh = plsc.VectorSubcoreMesh(core_axis_name="core",
                                  subcore_axis_name="subcore")

    @pl.kernel(out_type=jax.ShapeDtypeStruct((n_idx, dim), h.dtype), mesh=mesh)
    def k(h_hbm, i_hbm, o_hbm):
        def body(i_vmem, o_vmem):
            pltpu.sync_copy(h_hbm.at[i_vmem.at[0]], o_vmem)

        pltpu.emit_pipeline(
            body,
            grid=(n_idx // _GW,),
            in_specs=[pl.BlockSpec((1, _GW), lambda i: (0, i))],
            out_specs=[pl.BlockSpec((_GW, dim), lambda i: (i, 0))],
            core_axis_name=("core", "subcore"),
            dimension_semantics=(pltpu.PARALLEL,),
        )(i_hbm, o_hbm)

    return k(h, idx_flat)


def _heads_body(g_ref, w_ref, h1_ref, xs_ref,
                wo1, wo2a, wo2b, bo2,
                n2w1, n2b1, n2w2, n2b2, n2w3, n2b3, n2w4, n2b4,
                n3w1h, n3w1i, n3b1, n3w2, n3b2, n3w3, n3b3, n3w4, n3b4,
                ids_ref, p4_ref):
    msg0 = g_ref[:, 0, :] * w_ref[:, 0:1]
    acc = msg0
    mx = msg0
    for j in range(1, _K):
        msg = g_ref[:, j, :] * w_ref[:, j:j + 1]
        acc = acc + msg
        mx = jnp.maximum(mx, msg)
    mean = acc * (1.0 / _K)
    h2 = (jnp.dot(h1_ref[...], wo1[...], preferred_element_type=jnp.float32)
          + jnp.dot(mean, wo2a[...], preferred_element_type=jnp.float32)
          + jnp.dot(mx, wo2b[...], preferred_element_type=jnp.float32)
          + bo2[...])
    h2 = _leaky(h2)
    t = _leaky(jnp.dot(h2, n2w1[...], preferred_element_type=jnp.float32) + n2b1[...])
    t = _leaky(jnp.dot(t, n2w2[...], preferred_element_type=jnp.float32) + n2b2[...])
    t = _leaky(jnp.dot(t, n2w3[...], preferred_element_type=jnp.float32) + n2b3[...])
    ids = jnp.dot(t, n2w4[...], preferred_element_type=jnp.float32) + n2b4[...]
    ids_ref[...] = ids
    u = _leaky(jnp.dot(h2, n3w1h[...], preferred_element_type=jnp.float32)
               + jnp.dot(ids, n3w1i[...], preferred_element_type=jnp.float32)
               + n3b1[...])
    u = _leaky(jnp.dot(u, n3w2[...], preferred_element_type=jnp.float32) + n3b2[...])
    u = _leaky(jnp.dot(u, n3w3[...], preferred_element_type=jnp.float32) + n3b3[...])
    p4_ref[...] = (jnp.dot(u, n3w4[...], preferred_element_type=jnp.float32)
                   + n3b4[...] + xs_ref[...])


def _run_heads(g3, w, h1, xs, conv, nn2, nn3):
    n = h1.shape[0]
    grid = n // _RB
    Ws, bs, Wh, bh, Wo1, Wo2, bo2 = conv
    full = lambda a: pl.BlockSpec(a.shape, lambda i: (0,) * a.ndim)
    row = lambda d: pl.BlockSpec((_RB, d), lambda i: (i, 0))
    wargs = [Wo1.T, Wo2.T[:32, :], Wo2.T[32:, :], bo2.reshape(1, -1)]
    for (W, b) in nn2:
        wargs += [W.T, b.reshape(1, -1)]
    (V1, c1), (V2, c2), (V3, c3), (V4, c4) = nn3
    wargs += [V1.T[:32, :], V1.T[32:, :], c1.reshape(1, -1),
              V2.T, c2.reshape(1, -1), V3.T, c3.reshape(1, -1),
              V4.T, c4.reshape(1, -1)]
    in_specs = [pl.BlockSpec((_RB, _K, 32), lambda i: (i, 0, 0)),
                row(_K), row(32), row(4)]
    in_specs += [full(a) for a in wargs]
    return pl.pallas_call(
        _heads_body,
        grid=(grid,),
        in_specs=in_specs,
        out_specs=[row(6), row(4)],
        out_shape=[jax.ShapeDtypeStruct((n, 6), jnp.float32),
                   jax.ShapeDtypeStruct((n, 4), jnp.float32)],
    )(g3, w, h1, xs, *wargs)


def kernel(x, nn1, conv, nn2, nn3):
    Ws, bs, Wh, bh, Wo1, Wo2, bo2 = conv
    h1, h, s = _run_mlp_proj(x, nn1, Ws, bs, Wh, bh)
    st = jnp.full((8, _NPAD), 1e30, dtype=jnp.float32)
    st = st.at[:, :_N].set(0.0)
    st = st.at[0:2, :_N].set(s.T)
    nbr, w = _run_knn(s, st)
    g = _sc_gather(h, nbr.reshape(1, _N * _K).astype(jnp.int32))
    g3 = g.reshape(_N, _K, 32)
    ids, p4 = _run_heads(g3, w, h1, x[:, 3:7], conv, nn2, nn3)
    return (ids, p4)


# speedup vs baseline: 8.9284x; 8.9284x over previous
import jax
import jax.numpy as jnp
from jax.experimental import pallas as pl
from jax.experimental.pallas import tpu as pltpu
from jax.experimental.pallas import tpu_sc as plsc

_NEG = 0.01
_N = 10000
_K = 16
_QB = 256
_NPAD = 10240
_RB = 1000
_GW = 128


def _leaky(v):
    return jnp.where(v >= 0, v, _NEG * v)


def _mlp_proj_body(x_ref, w1, b1, w2, b2, w3, b3, w4, b4, wst, bs, wht, bh,
                   h1_ref, h_ref, s_ref):
    h = _leaky(jnp.dot(x_ref[...], w1[...],
                       preferred_element_type=jnp.float32) + b1[...])
    h = _leaky(jnp.dot(h, w2[...], preferred_element_type=jnp.float32) + b2[...])
    h = _leaky(jnp.dot(h, w3[...], preferred_element_type=jnp.float32) + b3[...])
    h = _leaky(jnp.dot(h, w4[...], preferred_element_type=jnp.float32) + b4[...])
    h1_ref[...] = h
    s_ref[...] = jnp.dot(h, wst[...], preferred_element_type=jnp.float32) + bs[...]
    h_ref[...] = jnp.dot(h, wht[...], preferred_element_type=jnp.float32) + bh[...]


def _run_mlp_proj(x, nn1, Ws, bs, Wh, bh):
    n = x.shape[0]
    grid = n // _RB
    full = lambda a: pl.BlockSpec(a.shape, lambda i: (0,) * a.ndim)
    row = lambda d: pl.BlockSpec((_RB, d), lambda i: (i, 0))
    wargs = []
    in_specs = [row(x.shape[1])]
    for (W, b) in nn1:
        wargs += [W.T, b.reshape(1, -1)]
    wht = jnp.zeros((32, 128), jnp.float32).at[:, :32].set(Wh.T)
    bht = jnp.zeros((1, 128), jnp.float32).at[:, :32].set(bh.reshape(1, -1))
    wargs += [Ws.T, bs.reshape(1, -1), wht, bht]
    in_specs += [full(a) for a in wargs]
    return pl.pallas_call(
        _mlp_proj_body,
        grid=(grid,),
        in_specs=in_specs,
        out_specs=[row(32), row(128), row(2)],
        out_shape=[jax.ShapeDtypeStruct((n, 32), jnp.float32),
                   jax.ShapeDtypeStruct((n, 128), jnp.float32),
                   jax.ShapeDtypeStruct((n, 2), jnp.float32)],
    )(x, *wargs)


_L = 512
_FC = _NPAD // _L
_INF = 3.0e38


def _knn_body(sq_ref, s0t_ref, s1t_ref, nbr_ref, w_ref,
              m1_ref, a1_ref, m2_ref, a2_ref):
    s0q = sq_ref[:, 0:1]
    s1q = sq_ref[:, 1:2]
    m1_ref[...] = jnp.full((_QB, _L), _INF, jnp.float32)
    m2_ref[...] = jnp.full((_QB, _L), _INF, jnp.float32)
    a1_ref[...] = jnp.zeros((_QB, _L), jnp.int32)
    a2_ref[...] = jnp.zeros((_QB, _L), jnp.int32)

    def fold(c, _):
        s0a = s0t_ref[pl.ds(c, 1), :]
        s1a = s1t_ref[pl.ds(c, 1), :]
        d0 = s0q - s0a
        d1 = s1q - s1a
        d2c = d0 * d0 + d1 * d1
        m1 = m1_ref[...]
        m2 = m2_ref[...]
        a1 = a1_ref[...]
        a2 = a2_ref[...]
        lt1 = d2c < m1
        lt2 = d2c < m2
        m2_ref[...] = jnp.where(lt1, m1, jnp.where(lt2, d2c, m2))
        a2_ref[...] = jnp.where(lt1, a1, jnp.where(lt2, c, a2))
        m1_ref[...] = jnp.where(lt1, d2c, m1)
        a1_ref[...] = jnp.where(lt1, c, a1)
        return 0

    jax.lax.fori_loop(0, _FC, fold, 0, unroll=False)

    lane = jax.lax.broadcasted_iota(jnp.int32, (_QB, _L), 1)
    lane_k = jax.lax.broadcasted_iota(jnp.int32, (_QB, _K), 1)

    def extract(t, carry):
        cols, mvs = carry
        m1 = m1_ref[...]
        mv = jnp.min(m1, axis=1)
        ml = jnp.argmin(m1, axis=1)
        hit = lane == ml[:, None]
        mc = jnp.max(jnp.where(hit, a1_ref[...], -1), axis=1)
        col = mc * _L + ml
        sel = lane_k == t
        cols = jnp.where(sel, col[:, None], cols)
        mvs = jnp.where(sel, mv[:, None], mvs)
        m1_ref[...] = jnp.where(hit, m2_ref[...], m1)
        a1_ref[...] = jnp.where(hit, a2_ref[...], a1_ref[...])
        m2_ref[...] = jnp.where(hit, _INF, m2_ref[...])
        return cols, mvs

    cols0 = jnp.zeros((_QB, _K), jnp.int32)
    mvs0 = jnp.zeros((_QB, _K), jnp.float32)
    cols, mvs = jax.lax.fori_loop(0, _K, extract, (cols0, mvs0), unroll=False)
    nbr_ref[...] = cols
    w_ref[...] = jnp.exp(-10.0 * mvs)


def _run_knn(s, s0t, s1t):
    grid = _NPAD // _QB
    return pl.pallas_call(
        _knn_body,
        grid=(grid,),
        in_specs=[pl.BlockSpec((_QB, 2), lambda i: (i, 0)),
                  pl.BlockSpec(s0t.shape, lambda i: (0, 0)),
                  pl.BlockSpec(s1t.shape, lambda i: (0, 0))],
        out_specs=[pl.BlockSpec((_QB, _K), lambda i: (i, 0)),
                   pl.BlockSpec((_QB, _K), lambda i: (i, 0))],
        out_shape=[jax.ShapeDtypeStruct((_N, _K), jnp.int32),
                   jax.ShapeDtypeStruct((_N, _K), jnp.float32)],
        scratch_shapes=[pltpu.VMEM((_QB, _L), jnp.float32),
                        pltpu.VMEM((_QB, _L), jnp.int32),
                        pltpu.VMEM((_QB, _L), jnp.float32),
                        pltpu.VMEM((_QB, _L), jnp.int32)],
    )(s, s0t, s1t)


def _sc_gather(h, idx_flat):
    n_idx = idx_flat.shape[1]
    dim = h.shape[1]
    mesh = plsc.VectorSubcoreMesh(core_axis_name="core",
                                  subcore_axis_name="subcore")

    @pl.kernel(out_type=jax.ShapeDtypeStruct((n_idx, dim), h.dtype), mesh=mesh)
    def k(h_hbm, i_hbm, o_hbm):
        def body(i_vmem, o_vmem):
            pltpu.sync_copy(h_hbm.at[i_vmem.at[0]], o_vmem)

        pltpu.emit_pipeline(
            body,
            grid=(n_idx // _GW,),
            in_specs=[pl.BlockSpec((1, _GW), lambda i: (0, i))],
            out_specs=[pl.BlockSpec((_GW, dim), lambda i: (i, 0))],
            core_axis_name=("core", "subcore"),
            dimension_semantics=(pltpu.PARALLEL,),
        )(i_hbm, o_hbm)

    return k(h, idx_flat)


def _heads_body(g_ref, w_ref, h1_ref, xs_ref,
                wo1, wo2a, wo2b, bo2,
                n2w1, n2b1, n2w2, n2b2, n2w3, n2b3, n2w4, n2b4,
                n3w1h, n3w1i, n3b1, n3w2, n3b2, n3w3, n3b3, n3w4, n3b4,
                ids_ref, p4_ref):
    msg0 = g_ref[:, 0, 0:32] * w_ref[:, 0:1]
    acc = msg0
    mx = msg0
    for j in range(1, _K):
        msg = g_ref[:, j, 0:32] * w_ref[:, j:j + 1]
        acc = acc + msg
        mx = jnp.maximum(mx, msg)
    mean = acc * (1.0 / _K)
    h2 = (jnp.dot(h1_ref[...], wo1[...], preferred_element_type=jnp.float32)
          + jnp.dot(mean, wo2a[...], preferred_element_type=jnp.float32)
          + jnp.dot(mx, wo2b[...], preferred_element_type=jnp.float32)
          + bo2[...])
    h2 = _leaky(h2)
    t = _leaky(jnp.dot(h2, n2w1[...], preferred_element_type=jnp.float32) + n2b1[...])
    t = _leaky(jnp.dot(t, n2w2[...], preferred_element_type=jnp.float32) + n2b2[...])
    t = _leaky(jnp.dot(t, n2w3[...], preferred_element_type=jnp.float32) + n2b3[...])
    ids = jnp.dot(t, n2w4[...], preferred_element_type=jnp.float32) + n2b4[...]
    ids_ref[...] = ids
    u = _leaky(jnp.dot(h2, n3w1h[...], preferred_element_type=jnp.float32)
               + jnp.dot(ids, n3w1i[...], preferred_element_type=jnp.float32)
               + n3b1[...])
    u = _leaky(jnp.dot(u, n3w2[...], preferred_element_type=jnp.float32) + n3b2[...])
    u = _leaky(jnp.dot(u, n3w3[...], preferred_element_type=jnp.float32) + n3b3[...])
    p4_ref[...] = (jnp.dot(u, n3w4[...], preferred_element_type=jnp.float32)
                   + n3b4[...] + xs_ref[...])


def _run_heads(g3, w, h1, xs, conv, nn2, nn3):
    n = h1.shape[0]
    grid = n // _RB
    Ws, bs, Wh, bh, Wo1, Wo2, bo2 = conv
    full = lambda a: pl.BlockSpec(a.shape, lambda i: (0,) * a.ndim)
    row = lambda d: pl.BlockSpec((_RB, d), lambda i: (i, 0))
    wargs = [Wo1.T, Wo2.T[:32, :], Wo2.T[32:, :], bo2.reshape(1, -1)]
    for (W, b) in nn2:
        wargs += [W.T, b.reshape(1, -1)]
    (V1, c1), (V2, c2), (V3, c3), (V4, c4) = nn3
    wargs += [V1.T[:32, :], V1.T[32:, :], c1.reshape(1, -1),
              V2.T, c2.reshape(1, -1), V3.T, c3.reshape(1, -1),
              V4.T, c4.reshape(1, -1)]
    in_specs = [pl.BlockSpec((_RB, _K, 128), lambda i: (i, 0, 0)),
                row(_K), row(32), row(4)]
    in_specs += [full(a) for a in wargs]
    return pl.pallas_call(
        _heads_body,
        grid=(grid,),
        in_specs=in_specs,
        out_specs=[row(6), row(4)],
        out_shape=[jax.ShapeDtypeStruct((n, 6), jnp.float32),
                   jax.ShapeDtypeStruct((n, 4), jnp.float32)],
    )(g3, w, h1, xs, *wargs)


def kernel(x, nn1, conv, nn2, nn3):
    Ws, bs, Wh, bh, Wo1, Wo2, bo2 = conv
    h1, h, s = _run_mlp_proj(x, nn1, Ws, bs, Wh, bh)
    pad = jnp.full((_NPAD - _N,), 1e30, dtype=jnp.float32)
    s0t = jnp.concatenate([s[:, 0], pad]).reshape(_FC, _L)
    s1t = jnp.concatenate([s[:, 1], pad]).reshape(_FC, _L)
    nbr, w = _run_knn(s, s0t, s1t)
    g = _sc_gather(h, nbr.reshape(1, _N * _K).astype(jnp.int32))
    g3 = g.reshape(_N, _K, 128)
    ids, p4 = _run_heads(g3, w, h1, x[:, 3:7], conv, nn2, nn3)
    return (ids, p4)


# speedup vs baseline: 10.2239x; 1.1451x over previous
import jax
import jax.numpy as jnp
from jax.experimental import pallas as pl
from jax.experimental.pallas import tpu as pltpu
from jax.experimental.pallas import tpu_sc as plsc

_NEG = 0.01
_N = 10000
_K = 16
_QB = 256
_NPAD = 10240
_RB = 1000
_GW = 128


def _leaky(v):
    return jnp.where(v >= 0, v, _NEG * v)


def _mlp_proj_body(x_ref, w1, b1, w2, b2, w3, b3, w4, b4, wst, bs, wht, bh,
                   h1_ref, h_ref, s_ref):
    h = _leaky(jnp.dot(x_ref[...], w1[...],
                       preferred_element_type=jnp.float32) + b1[...])
    h = _leaky(jnp.dot(h, w2[...], preferred_element_type=jnp.float32) + b2[...])
    h = _leaky(jnp.dot(h, w3[...], preferred_element_type=jnp.float32) + b3[...])
    h = _leaky(jnp.dot(h, w4[...], preferred_element_type=jnp.float32) + b4[...])
    h1_ref[...] = h
    s_ref[...] = jnp.dot(h, wst[...], preferred_element_type=jnp.float32) + bs[...]
    h_ref[...] = jnp.dot(h, wht[...], preferred_element_type=jnp.float32) + bh[...]


def _run_mlp_proj(x, nn1, Ws, bs, Wh, bh):
    n = x.shape[0]
    grid = n // _RB
    full = lambda a: pl.BlockSpec(a.shape, lambda i: (0,) * a.ndim)
    row = lambda d: pl.BlockSpec((_RB, d), lambda i: (i, 0))
    wargs = []
    in_specs = [row(x.shape[1])]
    for (W, b) in nn1:
        wargs += [W.T, b.reshape(1, -1)]
    wht = jnp.zeros((32, 128), jnp.float32).at[:, :32].set(Wh.T)
    bht = jnp.zeros((1, 128), jnp.float32).at[:, :32].set(bh.reshape(1, -1))
    wargs += [Ws.T, bs.reshape(1, -1), wht, bht]
    in_specs += [full(a) for a in wargs]
    return pl.pallas_call(
        _mlp_proj_body,
        grid=(grid,),
        in_specs=in_specs,
        out_specs=[row(32), row(128), row(2)],
        out_shape=[jax.ShapeDtypeStruct((n, 32), jnp.float32),
                   jax.ShapeDtypeStruct((n, 128), jnp.float32),
                   jax.ShapeDtypeStruct((n, 2), jnp.float32)],
    )(x, *wargs)


_L = 256
_FC = _NPAD // _L
_CMASK = 63


def _knn_body(sq_ref, s0t_ref, s1t_ref, nbr_ref, w_ref, m1_ref, m2_ref):
    s0q = sq_ref[:, 0:1]
    s1q = sq_ref[:, 1:2]
    big = jax.lax.bitcast_convert_type(
        jnp.full((_QB, _L), 3.0e38, jnp.float32), jnp.int32)
    m1_ref[...] = big
    m2_ref[...] = big

    def key_for(c):
        s0a = s0t_ref[pl.ds(c, 1), :]
        s1a = s1t_ref[pl.ds(c, 1), :]
        d0 = s0q - s0a
        d1 = s1q - s1a
        d2c = d0 * d0 + d1 * d1
        kb = jax.lax.bitcast_convert_type(d2c, jnp.int32)
        return (kb & ~_CMASK) | c

    def fold4(i, _):
        m1 = m1_ref[...]
        m2 = m2_ref[...]
        for dc in range(4):
            k = key_for(i * 4 + dc)
            m2 = jnp.minimum(m2, jnp.maximum(m1, k))
            m1 = jnp.minimum(m1, k)
        m1_ref[...] = m1
        m2_ref[...] = m2
        return 0

    jax.lax.fori_loop(0, _FC // 4, fold4, 0, unroll=False)

    lane = jax.lax.broadcasted_iota(jnp.int32, (_QB, _L), 1)
    lane_k = jax.lax.broadcasted_iota(jnp.int32, (_QB, _K), 1)

    def extract(t, carry):
        cols, kd2 = carry
        m1 = m1_ref[...]
        mf = jax.lax.bitcast_convert_type(m1, jnp.float32)
        kmin = jax.lax.bitcast_convert_type(jnp.min(mf, axis=1), jnp.int32)
        ml = jnp.argmin(mf, axis=1)
        hit = lane == ml[:, None]
        sel = lane_k == t
        cols = jnp.where(sel, ((kmin & _CMASK) * _L + ml)[:, None], cols)
        kd2 = jnp.where(sel, (kmin & ~_CMASK)[:, None], kd2)
        m1_ref[...] = jnp.where(hit, m2_ref[...], m1)
        m2_ref[...] = jnp.where(hit, big, m2_ref[...])
        return cols, kd2

    cols0 = jnp.zeros((_QB, _K), jnp.int32)
    kd20 = jnp.zeros((_QB, _K), jnp.int32)
    cols, kd2 = jax.lax.fori_loop(0, _K, extract, (cols0, kd20), unroll=False)
    nbr_ref[...] = cols
    w_ref[...] = jnp.exp(-10.0 * jax.lax.bitcast_convert_type(kd2, jnp.float32))


def _run_knn(s, s0t, s1t):
    grid = _NPAD // _QB
    return pl.pallas_call(
        _knn_body,
        grid=(grid,),
        in_specs=[pl.BlockSpec((_QB, 2), lambda i: (i, 0)),
                  pl.BlockSpec(s0t.shape, lambda i: (0, 0)),
                  pl.BlockSpec(s1t.shape, lambda i: (0, 0))],
        out_specs=[pl.BlockSpec((_QB, _K), lambda i: (i, 0)),
                   pl.BlockSpec((_QB, _K), lambda i: (i, 0))],
        out_shape=[jax.ShapeDtypeStruct((_N, _K), jnp.int32),
                   jax.ShapeDtypeStruct((_N, _K), jnp.float32)],
        scratch_shapes=[pltpu.VMEM((_QB, _L), jnp.int32),
                        pltpu.VMEM((_QB, _L), jnp.int32)],
    )(s, s0t, s1t)


def _sc_gather(h, idx_flat):
    n_idx = idx_flat.shape[1]
    dim = h.shape[1]
    mesh = plsc.VectorSubcoreMesh(core_axis_name="core",
                                  subcore_axis_name="subcore")

    @pl.kernel(out_type=jax.ShapeDtypeStruct((n_idx, dim), h.dtype), mesh=mesh)
    def k(h_hbm, i_hbm, o_hbm):
        def body(i_vmem, o_vmem):
            pltpu.sync_copy(h_hbm.at[i_vmem.at[0]], o_vmem)

        pltpu.emit_pipeline(
            body,
            grid=(n_idx // _GW,),
            in_specs=[pl.BlockSpec((1, _GW), lambda i: (0, i))],
            out_specs=[pl.BlockSpec((_GW, dim), lambda i: (i, 0))],
            core_axis_name=("core", "subcore"),
            dimension_semantics=(pltpu.PARALLEL,),
        )(i_hbm, o_hbm)

    return k(h, idx_flat)


def _heads_body(g_ref, w_ref, h1_ref, xs_ref,
                wo1, wo2a, wo2b, bo2,
                n2w1, n2b1, n2w2, n2b2, n2w3, n2b3, n2w4, n2b4,
                n3w1h, n3w1i, n3b1, n3w2, n3b2, n3w3, n3b3, n3w4, n3b4,
                ids_ref, p4_ref):
    msg0 = g_ref[:, 0, 0:32] * w_ref[:, 0:1]
    acc = msg0
    mx = msg0
    for j in range(1, _K):
        msg = g_ref[:, j, 0:32] * w_ref[:, j:j + 1]
        acc = acc + msg
        mx = jnp.maximum(mx, msg)
    mean = acc * (1.0 / _K)
    h2 = (jnp.dot(h1_ref[...], wo1[...], preferred_element_type=jnp.float32)
          + jnp.dot(mean, wo2a[...], preferred_element_type=jnp.float32)
          + jnp.dot(mx, wo2b[...], preferred_element_type=jnp.float32)
          + bo2[...])
    h2 = _leaky(h2)
    t = _leaky(jnp.dot(h2, n2w1[...], preferred_element_type=jnp.float32) + n2b1[...])
    t = _leaky(jnp.dot(t, n2w2[...], preferred_element_type=jnp.float32) + n2b2[...])
    t = _leaky(jnp.dot(t, n2w3[...], preferred_element_type=jnp.float32) + n2b3[...])
    ids = jnp.dot(t, n2w4[...], preferred_element_type=jnp.float32) + n2b4[...]
    ids_ref[...] = ids
    u = _leaky(jnp.dot(h2, n3w1h[...], preferred_element_type=jnp.float32)
               + jnp.dot(ids, n3w1i[...], preferred_element_type=jnp.float32)
               + n3b1[...])
    u = _leaky(jnp.dot(u, n3w2[...], preferred_element_type=jnp.float32) + n3b2[...])
    u = _leaky(jnp.dot(u, n3w3[...], preferred_element_type=jnp.float32) + n3b3[...])
    p4_ref[...] = (jnp.dot(u, n3w4[...], preferred_element_type=jnp.float32)
                   + n3b4[...] + xs_ref[...])


def _run_heads(g3, w, h1, xs, conv, nn2, nn3):
    n = h1.shape[0]
    grid = n // _RB
    Ws, bs, Wh, bh, Wo1, Wo2, bo2 = conv
    full = lambda a: pl.BlockSpec(a.shape, lambda i: (0,) * a.ndim)
    row = lambda d: pl.BlockSpec((_RB, d), lambda i: (i, 0))
    wargs = [Wo1.T, Wo2.T[:32, :], Wo2.T[32:, :], bo2.reshape(1, -1)]
    for (W, b) in nn2:
        wargs += [W.T, b.reshape(1, -1)]
    (V1, c1), (V2, c2), (V3, c3), (V4, c4) = nn3
    wargs += [V1.T[:32, :], V1.T[32:, :], c1.reshape(1, -1),
              V2.T, c2.reshape(1, -1), V3.T, c3.reshape(1, -1),
              V4.T, c4.reshape(1, -1)]
    in_specs = [pl.BlockSpec((_RB, _K, 128), lambda i: (i, 0, 0)),
                row(_K), row(32), row(4)]
    in_specs += [full(a) for a in wargs]
    return pl.pallas_call(
        _heads_body,
        grid=(grid,),
        in_specs=in_specs,
        out_specs=[row(6), row(4)],
        out_shape=[jax.ShapeDtypeStruct((n, 6), jnp.float32),
                   jax.ShapeDtypeStruct((n, 4), jnp.float32)],
    )(g3, w, h1, xs, *wargs)


def kernel(x, nn1, conv, nn2, nn3):
    Ws, bs, Wh, bh, Wo1, Wo2, bo2 = conv
    h1, h, s = _run_mlp_proj(x, nn1, Ws, bs, Wh, bh)
    pad = jnp.full((_NPAD - _N,), 1e18, dtype=jnp.float32)
    s0t = jnp.concatenate([s[:, 0], pad]).reshape(_FC, _L)
    s1t = jnp.concatenate([s[:, 1], pad]).reshape(_FC, _L)
    nbr, w = _run_knn(s, s0t, s1t)
    g = _sc_gather(h, nbr.reshape(1, _N * _K).astype(jnp.int32))
    g3 = g.reshape(_N, _K, 128)
    ids, p4 = _run_heads(g3, w, h1, x[:, 3:7], conv, nn2, nn3)
    return (ids, p4)


# speedup vs baseline: 10.6069x; 1.0375x over previous
import jax
import jax.numpy as jnp
from jax.experimental import pallas as pl
from jax.experimental.pallas import tpu as pltpu
from jax.experimental.pallas import tpu_sc as plsc

_NEG = 0.01
_N = 10000
_K = 16
_QB = 256
_NPAD = 10240
_RB = 1000
_GW = 128


def _leaky(v):
    return jnp.where(v >= 0, v, _NEG * v)


def _mlp_proj_body(x_ref, w1, b1, w2, b2, w3, b3, w4, b4, wst, bs, wht, bh,
                   h1_ref, h_ref, s_ref):
    h = _leaky(jnp.dot(x_ref[...], w1[...],
                       preferred_element_type=jnp.float32) + b1[...])
    h = _leaky(jnp.dot(h, w2[...], preferred_element_type=jnp.float32) + b2[...])
    h = _leaky(jnp.dot(h, w3[...], preferred_element_type=jnp.float32) + b3[...])
    h = _leaky(jnp.dot(h, w4[...], preferred_element_type=jnp.float32) + b4[...])
    h1_ref[...] = h
    s_ref[...] = jnp.dot(h, wst[...], preferred_element_type=jnp.float32) + bs[...]
    h_ref[...] = jnp.dot(h, wht[...], preferred_element_type=jnp.float32) + bh[...]


def _run_mlp_proj(x, nn1, Ws, bs, Wh, bh):
    n = x.shape[0]
    grid = n // _RB
    full = lambda a: pl.BlockSpec(a.shape, lambda i: (0,) * a.ndim)
    row = lambda d: pl.BlockSpec((_RB, d), lambda i: (i, 0))
    wargs = []
    in_specs = [row(x.shape[1])]
    for (W, b) in nn1:
        wargs += [W.T, b.reshape(1, -1)]
    wht = jnp.zeros((32, 128), jnp.float32).at[:, :32].set(Wh.T)
    bht = jnp.zeros((1, 128), jnp.float32).at[:, :32].set(bh.reshape(1, -1))
    wargs += [Ws.T, bs.reshape(1, -1), wht, bht]
    in_specs += [full(a) for a in wargs]
    return pl.pallas_call(
        _mlp_proj_body,
        grid=(grid,),
        in_specs=in_specs,
        out_specs=[row(32), row(128), row(2)],
        out_shape=[jax.ShapeDtypeStruct((n, 32), jnp.float32),
                   jax.ShapeDtypeStruct((n, 128), jnp.float32),
                   jax.ShapeDtypeStruct((n, 2), jnp.float32)],
    )(x, *wargs)


_L = 256
_FC = _NPAD // _L
_CMASK = 63


def _knn_body(sq_ref, s0t_ref, s1t_ref, nbr_ref, w_ref, m1_ref, m2_ref):
    s0q = sq_ref[:, 0:1]
    s1q = sq_ref[:, 1:2]
    big = jnp.full((_QB, _L), 3.0e38, jnp.float32)
    m1_ref[...] = big
    m2_ref[...] = big

    def key_for(c):
        s0a = s0t_ref[pl.ds(c, 1), :]
        s1a = s1t_ref[pl.ds(c, 1), :]
        d0 = s0q - s0a
        d1 = s1q - s1a
        d2c = d0 * d0 + d1 * d1
        kb = jax.lax.bitcast_convert_type(d2c, jnp.int32)
        return jax.lax.bitcast_convert_type((kb & ~_CMASK) | c, jnp.float32)

    def fold4(i, _):
        m1 = m1_ref[...]
        m2 = m2_ref[...]
        for dc in range(4):
            k = key_for(i * 4 + dc)
            m2 = jnp.minimum(m2, jnp.maximum(m1, k))
            m1 = jnp.minimum(m1, k)
        m1_ref[...] = m1
        m2_ref[...] = m2
        return 0

    jax.lax.fori_loop(0, _FC // 4, fold4, 0, unroll=False)

    lane = jax.lax.broadcasted_iota(jnp.int32, (_QB, _L), 1)
    lane_k = jax.lax.broadcasted_iota(jnp.int32, (_QB, _K), 1)

    def extract(t, carry):
        cols, kd2 = carry
        mf = m1_ref[...]
        kmin = jax.lax.bitcast_convert_type(jnp.min(mf, axis=1), jnp.int32)
        ml = jnp.argmin(mf, axis=1)
        hit = lane == ml[:, None]
        sel = lane_k == t
        cols = jnp.where(sel, ((kmin & _CMASK) * _L + ml)[:, None], cols)
        kd2 = jnp.where(sel, (kmin & ~_CMASK)[:, None], kd2)
        m1_ref[...] = jnp.where(hit, m2_ref[...], mf)
        m2_ref[...] = jnp.where(hit, big, m2_ref[...])
        return cols, kd2

    cols0 = jnp.zeros((_QB, _K), jnp.int32)
    kd20 = jnp.zeros((_QB, _K), jnp.int32)
    cols, kd2 = jax.lax.fori_loop(0, _K, extract, (cols0, kd20), unroll=False)
    nbr_ref[...] = cols
    w_ref[...] = jnp.exp(-10.0 * jax.lax.bitcast_convert_type(kd2, jnp.float32))


def _run_knn(s, s0t, s1t):
    grid = _NPAD // _QB
    return pl.pallas_call(
        _knn_body,
        grid=(grid,),
        in_specs=[pl.BlockSpec((_QB, 2), lambda i: (i, 0)),
                  pl.BlockSpec(s0t.shape, lambda i: (0, 0)),
                  pl.BlockSpec(s1t.shape, lambda i: (0, 0))],
        out_specs=[pl.BlockSpec((_QB, _K), lambda i: (i, 0)),
                   pl.BlockSpec((_QB, _K), lambda i: (i, 0))],
        out_shape=[jax.ShapeDtypeStruct((_N, _K), jnp.int32),
                   jax.ShapeDtypeStruct((_N, _K), jnp.float32)],
        scratch_shapes=[pltpu.VMEM((_QB, _L), jnp.float32),
                        pltpu.VMEM((_QB, _L), jnp.float32)],
    )(s, s0t, s1t)


def _sc_gather(h, idx_flat):
    n_idx = idx_flat.shape[1]
    dim = h.shape[1]
    mesh = plsc.VectorSubcoreMesh(core_axis_name="core",
                                  subcore_axis_name="subcore")

    @pl.kernel(out_type=jax.ShapeDtypeStruct((n_idx, dim), h.dtype), mesh=mesh)
    def k(h_hbm, i_hbm, o_hbm):
        def body(i_vmem, o_vmem):
            pltpu.sync_copy(h_hbm.at[i_vmem.at[0]], o_vmem)

        pltpu.emit_pipeline(
            body,
            grid=(n_idx // _GW,),
            in_specs=[pl.BlockSpec((1, _GW), lambda i: (0, i))],
            out_specs=[pl.BlockSpec((_GW, dim), lambda i: (i, 0))],
            core_axis_name=("core", "subcore"),
            dimension_semantics=(pltpu.PARALLEL,),
        )(i_hbm, o_hbm)

    return k(h, idx_flat)


def _heads_body(g_ref, w_ref, h1_ref, xs_ref,
                wo1, wo2a, wo2b, bo2,
                n2w1, n2b1, n2w2, n2b2, n2w3, n2b3, n2w4, n2b4,
                n3w1h, n3w1i, n3b1, n3w2, n3b2, n3w3, n3b3, n3w4, n3b4,
                ids_ref, p4_ref):
    msg0 = g_ref[:, 0, 0:32] * w_ref[:, 0:1]
    acc = msg0
    mx = msg0
    for j in range(1, _K):
        msg = g_ref[:, j, 0:32] * w_ref[:, j:j + 1]
        acc = acc + msg
        mx = jnp.maximum(mx, msg)
    mean = acc * (1.0 / _K)
    h2 = (jnp.dot(h1_ref[...], wo1[...], preferred_element_type=jnp.float32)
          + jnp.dot(mean, wo2a[...], preferred_element_type=jnp.float32)
          + jnp.dot(mx, wo2b[...], preferred_element_type=jnp.float32)
          + bo2[...])
    h2 = _leaky(h2)
    t = _leaky(jnp.dot(h2, n2w1[...], preferred_element_type=jnp.float32) + n2b1[...])
    t = _leaky(jnp.dot(t, n2w2[...], preferred_element_type=jnp.float32) + n2b2[...])
    t = _leaky(jnp.dot(t, n2w3[...], preferred_element_type=jnp.float32) + n2b3[...])
    ids = jnp.dot(t, n2w4[...], preferred_element_type=jnp.float32) + n2b4[...]
    ids_ref[...] = ids
    u = _leaky(jnp.dot(h2, n3w1h[...], preferred_element_type=jnp.float32)
               + jnp.dot(ids, n3w1i[...], preferred_element_type=jnp.float32)
               + n3b1[...])
    u = _leaky(jnp.dot(u, n3w2[...], preferred_element_type=jnp.float32) + n3b2[...])
    u = _leaky(jnp.dot(u, n3w3[...], preferred_element_type=jnp.float32) + n3b3[...])
    p4_ref[...] = (jnp.dot(u, n3w4[...], preferred_element_type=jnp.float32)
                   + n3b4[...] + xs_ref[...])


def _run_heads(g3, w, h1, xs, conv, nn2, nn3):
    n = h1.shape[0]
    grid = n // _RB
    Ws, bs, Wh, bh, Wo1, Wo2, bo2 = conv
    full = lambda a: pl.BlockSpec(a.shape, lambda i: (0,) * a.ndim)
    row = lambda d: pl.BlockSpec((_RB, d), lambda i: (i, 0))
    wargs = [Wo1.T, Wo2.T[:32, :], Wo2.T[32:, :], bo2.reshape(1, -1)]
    for (W, b) in nn2:
        wargs += [W.T, b.reshape(1, -1)]
    (V1, c1), (V2, c2), (V3, c3), (V4, c4) = nn3
    wargs += [V1.T[:32, :], V1.T[32:, :], c1.reshape(1, -1),
              V2.T, c2.reshape(1, -1), V3.T, c3.reshape(1, -1),
              V4.T, c4.reshape(1, -1)]
    in_specs = [pl.BlockSpec((_RB, _K, 128), lambda i: (i, 0, 0)),
                row(_K), row(32), row(4)]
    in_specs += [full(a) for a in wargs]
    return pl.pallas_call(
        _heads_body,
        grid=(grid,),
        in_specs=in_specs,
        out_specs=[row(6), row(4)],
        out_shape=[jax.ShapeDtypeStruct((n, 6), jnp.float32),
                   jax.ShapeDtypeStruct((n, 4), jnp.float32)],
    )(g3, w, h1, xs, *wargs)


def kernel(x, nn1, conv, nn2, nn3):
    Ws, bs, Wh, bh, Wo1, Wo2, bo2 = conv
    h1, h, s = _run_mlp_proj(x, nn1, Ws, bs, Wh, bh)
    pad = jnp.full((_NPAD - _N,), 1e18, dtype=jnp.float32)
    s0t = jnp.concatenate([s[:, 0], pad]).reshape(_FC, _L)
    s1t = jnp.concatenate([s[:, 1], pad]).reshape(_FC, _L)
    nbr, w = _run_knn(s, s0t, s1t)
    g = _sc_gather(h, nbr.reshape(1, _N * _K).astype(jnp.int32))
    g3 = g.reshape(_N, _K, 128)
    ids, p4 = _run_heads(g3, w, h1, x[:, 3:7], conv, nn2, nn3)
    return (ids, p4)


# speedup vs baseline: 13.4536x; 1.2684x over previous
import jax
import jax.numpy as jnp
from jax.experimental import pallas as pl
from jax.experimental.pallas import tpu as pltpu
from jax.experimental.pallas import tpu_sc as plsc

_NEG = 0.01
_N = 10000
_K = 16
_QB = 512
_NPAD = 10240
_RB = 1000
_GW = 128


def _leaky(v):
    return jnp.where(v >= 0, v, _NEG * v)


def _mlp_proj_body(x_ref, w1, b1, w2, b2, w3, b3, w4, b4, wst, bs, wht, bh,
                   h1_ref, h_ref, s_ref):
    h = _leaky(jnp.dot(x_ref[...], w1[...],
                       preferred_element_type=jnp.float32) + b1[...])
    h = _leaky(jnp.dot(h, w2[...], preferred_element_type=jnp.float32) + b2[...])
    h = _leaky(jnp.dot(h, w3[...], preferred_element_type=jnp.float32) + b3[...])
    h = _leaky(jnp.dot(h, w4[...], preferred_element_type=jnp.float32) + b4[...])
    h1_ref[...] = h
    s_ref[...] = jnp.dot(h, wst[...], preferred_element_type=jnp.float32) + bs[...]
    h_ref[...] = jnp.dot(h, wht[...], preferred_element_type=jnp.float32) + bh[...]


def _run_mlp_proj(x, nn1, Ws, bs, Wh, bh):
    n = x.shape[0]
    grid = n // _RB
    full = lambda a: pl.BlockSpec(a.shape, lambda i: (0,) * a.ndim)
    row = lambda d: pl.BlockSpec((_RB, d), lambda i: (i, 0))
    wargs = []
    in_specs = [row(x.shape[1])]
    for (W, b) in nn1:
        wargs += [W.T, b.reshape(1, -1)]
    wht = jnp.zeros((32, 128), jnp.float32).at[:, :32].set(Wh.T)
    bht = jnp.zeros((1, 128), jnp.float32).at[:, :32].set(bh.reshape(1, -1))
    wargs += [Ws.T, bs.reshape(1, -1), wht, bht]
    in_specs += [full(a) for a in wargs]
    return pl.pallas_call(
        _mlp_proj_body,
        grid=(grid,),
        in_specs=in_specs,
        out_specs=[row(32), row(128), row(2)],
        out_shape=[jax.ShapeDtypeStruct((n, 32), jnp.float32),
                   jax.ShapeDtypeStruct((n, 128), jnp.float32),
                   jax.ShapeDtypeStruct((n, 2), jnp.float32)],
    )(x, *wargs)


_L = 256
_FC = _NPAD // _L
_CMASK = 63


def _knn_body(sq_ref, s0t_ref, s1t_ref, nbr_ref, w_ref, m1_ref, m2_ref):
    s0q = sq_ref[:, 0:1]
    s1q = sq_ref[:, 1:2]
    big = jnp.full((_QB, _L), 3.0e38, jnp.float32)
    m1_ref[...] = big
    m2_ref[...] = big

    def key_for(c):
        s0a = s0t_ref[pl.ds(c, 1), :]
        s1a = s1t_ref[pl.ds(c, 1), :]
        d0 = s0q - s0a
        d1 = s1q - s1a
        d2c = d0 * d0 + d1 * d1
        kb = jax.lax.bitcast_convert_type(d2c, jnp.int32)
        return jax.lax.bitcast_convert_type((kb & ~_CMASK) | c, jnp.float32)

    def fold4(i, _):
        m1 = m1_ref[...]
        m2 = m2_ref[...]
        for dc in range(4):
            k = key_for(i * 4 + dc)
            m2 = jnp.minimum(m2, jnp.maximum(m1, k))
            m1 = jnp.minimum(m1, k)
        m1_ref[...] = m1
        m2_ref[...] = m2
        return 0

    jax.lax.fori_loop(0, _FC // 4, fold4, 0, unroll=False)

    lane = jax.lax.broadcasted_iota(jnp.int32, (_QB, _L), 1)
    lane_k = jax.lax.broadcasted_iota(jnp.int32, (_QB, _K), 1)

    def extract(t, carry):
        cols, kd2 = carry
        mf = m1_ref[...]
        kmin = jax.lax.bitcast_convert_type(jnp.min(mf, axis=1), jnp.int32)
        ml = jnp.argmin(mf, axis=1)
        hit = lane == ml[:, None]
        sel = lane_k == t
        cols = jnp.where(sel, ((kmin & _CMASK) * _L + ml)[:, None], cols)
        kd2 = jnp.where(sel, (kmin & ~_CMASK)[:, None], kd2)
        m1_ref[...] = jnp.where(hit, m2_ref[...], mf)
        m2_ref[...] = jnp.where(hit, big, m2_ref[...])
        return cols, kd2

    cols0 = jnp.zeros((_QB, _K), jnp.int32)
    kd20 = jnp.zeros((_QB, _K), jnp.int32)
    cols, kd2 = jax.lax.fori_loop(0, _K, extract, (cols0, kd20), unroll=False)
    nbr_ref[...] = cols
    w_ref[...] = jnp.exp(-10.0 * jax.lax.bitcast_convert_type(kd2, jnp.float32))


def _run_knn(s, s0t, s1t):
    grid = _NPAD // _QB
    return pl.pallas_call(
        _knn_body,
        grid=(grid,),
        in_specs=[pl.BlockSpec((_QB, 2), lambda i: (i, 0)),
                  pl.BlockSpec(s0t.shape, lambda i: (0, 0)),
                  pl.BlockSpec(s1t.shape, lambda i: (0, 0))],
        out_specs=[pl.BlockSpec((_QB, _K), lambda i: (i, 0)),
                   pl.BlockSpec((_QB, _K), lambda i: (i, 0))],
        out_shape=[jax.ShapeDtypeStruct((_N, _K), jnp.int32),
                   jax.ShapeDtypeStruct((_N, _K), jnp.float32)],
        scratch_shapes=[pltpu.VMEM((_QB, _L), jnp.float32),
                        pltpu.VMEM((_QB, _L), jnp.float32)],
    )(s, s0t, s1t)


def _sc_gather(h, idx_flat):
    n_idx = idx_flat.shape[1]
    dim = h.shape[1]
    mesh = plsc.VectorSubcoreMesh(core_axis_name="core",
                                  subcore_axis_name="subcore")

    @pl.kernel(out_type=jax.ShapeDtypeStruct((n_idx, dim), h.dtype), mesh=mesh)
    def k(h_hbm, i_hbm, o_hbm):
        def body(i_vmem, o_vmem):
            pltpu.sync_copy(h_hbm.at[i_vmem.at[0]], o_vmem)

        pltpu.emit_pipeline(
            body,
            grid=(n_idx // _GW,),
            in_specs=[pl.BlockSpec((1, _GW), lambda i: (0, i))],
            out_specs=[pl.BlockSpec((_GW, dim), lambda i: (i, 0))],
            core_axis_name=("core", "subcore"),
            dimension_semantics=(pltpu.PARALLEL,),
        )(i_hbm, o_hbm)

    return k(h, idx_flat)


def _heads_body(g_ref, w_ref, h1_ref, xs_ref,
                wo1, wo2a, wo2b, bo2,
                n2w1, n2b1, n2w2, n2b2, n2w3, n2b3, n2w4, n2b4,
                n3w1h, n3w1i, n3b1, n3w2, n3b2, n3w3, n3b3, n3w4, n3b4,
                ids_ref, p4_ref):
    msg0 = g_ref[:, 0, 0:32] * w_ref[:, 0:1]
    acc = msg0
    mx = msg0
    for j in range(1, _K):
        msg = g_ref[:, j, 0:32] * w_ref[:, j:j + 1]
        acc = acc + msg
        mx = jnp.maximum(mx, msg)
    mean = acc * (1.0 / _K)
    h2 = (jnp.dot(h1_ref[...], wo1[...], preferred_element_type=jnp.float32)
          + jnp.dot(mean, wo2a[...], preferred_element_type=jnp.float32)
          + jnp.dot(mx, wo2b[...], preferred_element_type=jnp.float32)
          + bo2[...])
    h2 = _leaky(h2)
    t = _leaky(jnp.dot(h2, n2w1[...], preferred_element_type=jnp.float32) + n2b1[...])
    t = _leaky(jnp.dot(t, n2w2[...], preferred_element_type=jnp.float32) + n2b2[...])
    t = _leaky(jnp.dot(t, n2w3[...], preferred_element_type=jnp.float32) + n2b3[...])
    ids = jnp.dot(t, n2w4[...], preferred_element_type=jnp.float32) + n2b4[...]
    ids_ref[...] = ids
    u = _leaky(jnp.dot(h2, n3w1h[...], preferred_element_type=jnp.float32)
               + jnp.dot(ids, n3w1i[...], preferred_element_type=jnp.float32)
               + n3b1[...])
    u = _leaky(jnp.dot(u, n3w2[...], preferred_element_type=jnp.float32) + n3b2[...])
    u = _leaky(jnp.dot(u, n3w3[...], preferred_element_type=jnp.float32) + n3b3[...])
    p4_ref[...] = (jnp.dot(u, n3w4[...], preferred_element_type=jnp.float32)
                   + n3b4[...] + xs_ref[...])


def _run_heads(g3, w, h1, xs, conv, nn2, nn3):
    n = h1.shape[0]
    grid = n // _RB
    Ws, bs, Wh, bh, Wo1, Wo2, bo2 = conv
    full = lambda a: pl.BlockSpec(a.shape, lambda i: (0,) * a.ndim)
    row = lambda d: pl.BlockSpec((_RB, d), lambda i: (i, 0))
    wargs = [Wo1.T, Wo2.T[:32, :], Wo2.T[32:, :], bo2.reshape(1, -1)]
    for (W, b) in nn2:
        wargs += [W.T, b.reshape(1, -1)]
    (V1, c1), (V2, c2), (V3, c3), (V4, c4) = nn3
    wargs += [V1.T[:32, :], V1.T[32:, :], c1.reshape(1, -1),
              V2.T, c2.reshape(1, -1), V3.T, c3.reshape(1, -1),
              V4.T, c4.reshape(1, -1)]
    in_specs = [pl.BlockSpec((_RB, _K, 128), lambda i: (i, 0, 0)),
                row(_K), row(32), row(4)]
    in_specs += [full(a) for a in wargs]
    return pl.pallas_call(
        _heads_body,
        grid=(grid,),
        in_specs=in_specs,
        out_specs=[row(6), row(4)],
        out_shape=[jax.ShapeDtypeStruct((n, 6), jnp.float32),
                   jax.ShapeDtypeStruct((n, 4), jnp.float32)],
    )(g3, w, h1, xs, *wargs)


def kernel(x, nn1, conv, nn2, nn3):
    Ws, bs, Wh, bh, Wo1, Wo2, bo2 = conv
    h1, h, s = _run_mlp_proj(x, nn1, Ws, bs, Wh, bh)
    pad = jnp.full((_NPAD - _N,), 1e18, dtype=jnp.float32)
    s0t = jnp.concatenate([s[:, 0], pad]).reshape(_FC, _L)
    s1t = jnp.concatenate([s[:, 1], pad]).reshape(_FC, _L)
    nbr, w = _run_knn(s, s0t, s1t)
    g = _sc_gather(h, nbr.reshape(1, _N * _K).astype(jnp.int32))
    g3 = g.reshape(_N, _K, 128)
    ids, p4 = _run_heads(g3, w, h1, x[:, 3:7], conv, nn2, nn3)
    return (ids, p4)


# speedup vs baseline: 16.0717x; 1.1946x over previous
import jax
import jax.numpy as jnp
from jax.experimental import pallas as pl
from jax.experimental.pallas import tpu as pltpu
from jax.experimental.pallas import tpu_sc as plsc

_NEG = 0.01
_N = 10000
_K = 16
_QB = 1024
_NPAD = 10240
_RB = 1000
_GW = 128


def _leaky(v):
    return jnp.where(v >= 0, v, _NEG * v)


def _mlp_proj_body(x_ref, w1, b1, w2, b2, w3, b3, w4, b4, wst, bs, wht, bh,
                   h1_ref, h_ref, s_ref):
    h = _leaky(jnp.dot(x_ref[...], w1[...],
                       preferred_element_type=jnp.float32) + b1[...])
    h = _leaky(jnp.dot(h, w2[...], preferred_element_type=jnp.float32) + b2[...])
    h = _leaky(jnp.dot(h, w3[...], preferred_element_type=jnp.float32) + b3[...])
    h = _leaky(jnp.dot(h, w4[...], preferred_element_type=jnp.float32) + b4[...])
    h1_ref[...] = h
    s_ref[...] = jnp.dot(h, wst[...], preferred_element_type=jnp.float32) + bs[...]
    h_ref[...] = jnp.dot(h, wht[...], preferred_element_type=jnp.float32) + bh[...]


def _run_mlp_proj(x, nn1, Ws, bs, Wh, bh):
    n = x.shape[0]
    grid = n // _RB
    full = lambda a: pl.BlockSpec(a.shape, lambda i: (0,) * a.ndim)
    row = lambda d: pl.BlockSpec((_RB, d), lambda i: (i, 0))
    wargs = []
    in_specs = [row(x.shape[1])]
    for (W, b) in nn1:
        wargs += [W.T, b.reshape(1, -1)]
    wht = jnp.zeros((32, 128), jnp.float32).at[:, :32].set(Wh.T)
    bht = jnp.zeros((1, 128), jnp.float32).at[:, :32].set(bh.reshape(1, -1))
    wargs += [Ws.T, bs.reshape(1, -1), wht, bht]
    in_specs += [full(a) for a in wargs]
    return pl.pallas_call(
        _mlp_proj_body,
        grid=(grid,),
        in_specs=in_specs,
        out_specs=[row(32), row(128), row(2)],
        out_shape=[jax.ShapeDtypeStruct((n, 32), jnp.float32),
                   jax.ShapeDtypeStruct((n, 128), jnp.float32),
                   jax.ShapeDtypeStruct((n, 2), jnp.float32)],
    )(x, *wargs)


_L = 256
_FC = _NPAD // _L
_CMASK = 63


def _knn_body(sq_ref, s0t_ref, s1t_ref, nbr_ref, w_ref, m1_ref, m2_ref):
    s0q = sq_ref[:, 0:1]
    s1q = sq_ref[:, 1:2]
    big = jnp.full((_QB, _L), 3.0e38, jnp.float32)
    m1_ref[...] = big
    m2_ref[...] = big

    def key_for(c):
        s0a = s0t_ref[pl.ds(c, 1), :]
        s1a = s1t_ref[pl.ds(c, 1), :]
        d0 = s0q - s0a
        d1 = s1q - s1a
        d2c = d0 * d0 + d1 * d1
        kb = jax.lax.bitcast_convert_type(d2c, jnp.int32)
        return jax.lax.bitcast_convert_type((kb & ~_CMASK) | c, jnp.float32)

    def fold4(i, _):
        m1 = m1_ref[...]
        m2 = m2_ref[...]
        for dc in range(4):
            k = key_for(i * 4 + dc)
            m2 = jnp.minimum(m2, jnp.maximum(m1, k))
            m1 = jnp.minimum(m1, k)
        m1_ref[...] = m1
        m2_ref[...] = m2
        return 0

    jax.lax.fori_loop(0, _FC // 4, fold4, 0, unroll=False)

    lane = jax.lax.broadcasted_iota(jnp.int32, (_QB, _L), 1)
    lane_k = jax.lax.broadcasted_iota(jnp.int32, (_QB, _K), 1)

    def extract(t, carry):
        cols, kd2 = carry
        mf = m1_ref[...]
        kmin = jax.lax.bitcast_convert_type(jnp.min(mf, axis=1), jnp.int32)
        ml = jnp.argmin(mf, axis=1)
        hit = lane == ml[:, None]
        sel = lane_k == t
        cols = jnp.where(sel, ((kmin & _CMASK) * _L + ml)[:, None], cols)
        kd2 = jnp.where(sel, (kmin & ~_CMASK)[:, None], kd2)
        m1_ref[...] = jnp.where(hit, m2_ref[...], mf)
        m2_ref[...] = jnp.where(hit, big, m2_ref[...])
        return cols, kd2

    cols0 = jnp.zeros((_QB, _K), jnp.int32)
    kd20 = jnp.zeros((_QB, _K), jnp.int32)
    cols, kd2 = jax.lax.fori_loop(0, _K, extract, (cols0, kd20), unroll=False)
    nbr_ref[...] = cols
    w_ref[...] = jnp.exp(-10.0 * jax.lax.bitcast_convert_type(kd2, jnp.float32))


def _run_knn(s, s0t, s1t):
    grid = _NPAD // _QB
    return pl.pallas_call(
        _knn_body,
        grid=(grid,),
        in_specs=[pl.BlockSpec((_QB, 2), lambda i: (i, 0)),
                  pl.BlockSpec(s0t.shape, lambda i: (0, 0)),
                  pl.BlockSpec(s1t.shape, lambda i: (0, 0))],
        out_specs=[pl.BlockSpec((_QB, _K), lambda i: (i, 0)),
                   pl.BlockSpec((_QB, _K), lambda i: (i, 0))],
        out_shape=[jax.ShapeDtypeStruct((_N, _K), jnp.int32),
                   jax.ShapeDtypeStruct((_N, _K), jnp.float32)],
        scratch_shapes=[pltpu.VMEM((_QB, _L), jnp.float32),
                        pltpu.VMEM((_QB, _L), jnp.float32)],
    )(s, s0t, s1t)


def _sc_gather(h, idx_flat):
    n_idx = idx_flat.shape[1]
    dim = h.shape[1]
    mesh = plsc.VectorSubcoreMesh(core_axis_name="core",
                                  subcore_axis_name="subcore")

    @pl.kernel(out_type=jax.ShapeDtypeStruct((n_idx, dim), h.dtype), mesh=mesh)
    def k(h_hbm, i_hbm, o_hbm):
        def body(i_vmem, o_vmem):
            pltpu.sync_copy(h_hbm.at[i_vmem.at[0]], o_vmem)

        pltpu.emit_pipeline(
            body,
            grid=(n_idx // _GW,),
            in_specs=[pl.BlockSpec((1, _GW), lambda i: (0, i))],
            out_specs=[pl.BlockSpec((_GW, dim), lambda i: (i, 0))],
            core_axis_name=("core", "subcore"),
            dimension_semantics=(pltpu.PARALLEL,),
        )(i_hbm, o_hbm)

    return k(h, idx_flat)


def _heads_body(g_ref, w_ref, h1_ref, xs_ref,
                wo1, wo2a, wo2b, bo2,
                n2w1, n2b1, n2w2, n2b2, n2w3, n2b3, n2w4, n2b4,
                n3w1h, n3w1i, n3b1, n3w2, n3b2, n3w3, n3b3, n3w4, n3b4,
                ids_ref, p4_ref):
    msg0 = g_ref[:, 0, 0:32] * w_ref[:, 0:1]
    acc = msg0
    mx = msg0
    for j in range(1, _K):
        msg = g_ref[:, j, 0:32] * w_ref[:, j:j + 1]
        acc = acc + msg
        mx = jnp.maximum(mx, msg)
    mean = acc * (1.0 / _K)
    h2 = (jnp.dot(h1_ref[...], wo1[...], preferred_element_type=jnp.float32)
          + jnp.dot(mean, wo2a[...], preferred_element_type=jnp.float32)
          + jnp.dot(mx, wo2b[...], preferred_element_type=jnp.float32)
          + bo2[...])
    h2 = _leaky(h2)
    t = _leaky(jnp.dot(h2, n2w1[...], preferred_element_type=jnp.float32) + n2b1[...])
    t = _leaky(jnp.dot(t, n2w2[...], preferred_element_type=jnp.float32) + n2b2[...])
    t = _leaky(jnp.dot(t, n2w3[...], preferred_element_type=jnp.float32) + n2b3[...])
    ids = jnp.dot(t, n2w4[...], preferred_element_type=jnp.float32) + n2b4[...]
    ids_ref[...] = ids
    u = _leaky(jnp.dot(h2, n3w1h[...], preferred_element_type=jnp.float32)
               + jnp.dot(ids, n3w1i[...], preferred_element_type=jnp.float32)
               + n3b1[...])
    u = _leaky(jnp.dot(u, n3w2[...], preferred_element_type=jnp.float32) + n3b2[...])
    u = _leaky(jnp.dot(u, n3w3[...], preferred_element_type=jnp.float32) + n3b3[...])
    p4_ref[...] = (jnp.dot(u, n3w4[...], preferred_element_type=jnp.float32)
                   + n3b4[...] + xs_ref[...])


def _run_heads(g3, w, h1, xs, conv, nn2, nn3):
    n = h1.shape[0]
    grid = n // _RB
    Ws, bs, Wh, bh, Wo1, Wo2, bo2 = conv
    full = lambda a: pl.BlockSpec(a.shape, lambda i: (0,) * a.ndim)
    row = lambda d: pl.BlockSpec((_RB, d), lambda i: (i, 0))
    wargs = [Wo1.T, Wo2.T[:32, :], Wo2.T[32:, :], bo2.reshape(1, -1)]
    for (W, b) in nn2:
        wargs += [W.T, b.reshape(1, -1)]
    (V1, c1), (V2, c2), (V3, c3), (V4, c4) = nn3
    wargs += [V1.T[:32, :], V1.T[32:, :], c1.reshape(1, -1),
              V2.T, c2.reshape(1, -1), V3.T, c3.reshape(1, -1),
              V4.T, c4.reshape(1, -1)]
    in_specs = [pl.BlockSpec((_RB, _K, 128), lambda i: (i, 0, 0)),
                row(_K), row(32), row(4)]
    in_specs += [full(a) for a in wargs]
    return pl.pallas_call(
        _heads_body,
        grid=(grid,),
        in_specs=in_specs,
        out_specs=[row(6), row(4)],
        out_shape=[jax.ShapeDtypeStruct((n, 6), jnp.float32),
                   jax.ShapeDtypeStruct((n, 4), jnp.float32)],
    )(g3, w, h1, xs, *wargs)


def kernel(x, nn1, conv, nn2, nn3):
    Ws, bs, Wh, bh, Wo1, Wo2, bo2 = conv
    h1, h, s = _run_mlp_proj(x, nn1, Ws, bs, Wh, bh)
    pad = jnp.full((_NPAD - _N,), 1e18, dtype=jnp.float32)
    s0t = jnp.concatenate([s[:, 0], pad]).reshape(_FC, _L)
    s1t = jnp.concatenate([s[:, 1], pad]).reshape(_FC, _L)
    nbr, w = _run_knn(s, s0t, s1t)
    g = _sc_gather(h, nbr.reshape(1, _N * _K).astype(jnp.int32))
    g3 = g.reshape(_N, _K, 128)
    ids, p4 = _run_heads(g3, w, h1, x[:, 3:7], conv, nn2, nn3)
    return (ids, p4)


# speedup vs baseline: 17.9069x; 1.1142x over previous
import jax
import jax.numpy as jnp
from jax.experimental import pallas as pl
from jax.experimental.pallas import tpu as pltpu
from jax.experimental.pallas import tpu_sc as plsc

_NEG = 0.01
_N = 10000
_K = 16
_QB = 2048
_NPAD = 10240
_RB = 1000
_GW = 128


def _leaky(v):
    return jnp.where(v >= 0, v, _NEG * v)


def _mlp_proj_body(x_ref, w1, b1, w2, b2, w3, b3, w4, b4, wst, bs, wht, bh,
                   h1_ref, h_ref, s_ref):
    h = _leaky(jnp.dot(x_ref[...], w1[...],
                       preferred_element_type=jnp.float32) + b1[...])
    h = _leaky(jnp.dot(h, w2[...], preferred_element_type=jnp.float32) + b2[...])
    h = _leaky(jnp.dot(h, w3[...], preferred_element_type=jnp.float32) + b3[...])
    h = _leaky(jnp.dot(h, w4[...], preferred_element_type=jnp.float32) + b4[...])
    h1_ref[...] = h
    s_ref[...] = jnp.dot(h, wst[...], preferred_element_type=jnp.float32) + bs[...]
    h_ref[...] = jnp.dot(h, wht[...], preferred_element_type=jnp.float32) + bh[...]


def _run_mlp_proj(x, nn1, Ws, bs, Wh, bh):
    n = x.shape[0]
    grid = n // _RB
    full = lambda a: pl.BlockSpec(a.shape, lambda i: (0,) * a.ndim)
    row = lambda d: pl.BlockSpec((_RB, d), lambda i: (i, 0))
    wargs = []
    in_specs = [row(x.shape[1])]
    for (W, b) in nn1:
        wargs += [W.T, b.reshape(1, -1)]
    wht = jnp.zeros((32, 128), jnp.float32).at[:, :32].set(Wh.T)
    bht = jnp.zeros((1, 128), jnp.float32).at[:, :32].set(bh.reshape(1, -1))
    wargs += [Ws.T, bs.reshape(1, -1), wht, bht]
    in_specs += [full(a) for a in wargs]
    return pl.pallas_call(
        _mlp_proj_body,
        grid=(grid,),
        in_specs=in_specs,
        out_specs=[row(32), row(128), row(2)],
        out_shape=[jax.ShapeDtypeStruct((n, 32), jnp.float32),
                   jax.ShapeDtypeStruct((n, 128), jnp.float32),
                   jax.ShapeDtypeStruct((n, 2), jnp.float32)],
    )(x, *wargs)


_L = 256
_FC = _NPAD // _L
_CMASK = 63


def _knn_body(sq_ref, s0t_ref, s1t_ref, nbr_ref, w_ref, m1_ref, m2_ref):
    s0q = sq_ref[:, 0:1]
    s1q = sq_ref[:, 1:2]
    big = jnp.full((_QB, _L), 3.0e38, jnp.float32)
    m1_ref[...] = big
    m2_ref[...] = big

    def key_for(c):
        s0a = s0t_ref[pl.ds(c, 1), :]
        s1a = s1t_ref[pl.ds(c, 1), :]
        d0 = s0q - s0a
        d1 = s1q - s1a
        d2c = d0 * d0 + d1 * d1
        kb = jax.lax.bitcast_convert_type(d2c, jnp.int32)
        return jax.lax.bitcast_convert_type((kb & ~_CMASK) | c, jnp.float32)

    def fold4(i, _):
        m1 = m1_ref[...]
        m2 = m2_ref[...]
        for dc in range(4):
            k = key_for(i * 4 + dc)
            m2 = jnp.minimum(m2, jnp.maximum(m1, k))
            m1 = jnp.minimum(m1, k)
        m1_ref[...] = m1
        m2_ref[...] = m2
        return 0

    jax.lax.fori_loop(0, _FC // 4, fold4, 0, unroll=False)

    lane = jax.lax.broadcasted_iota(jnp.int32, (_QB, _L), 1)
    lane_k = jax.lax.broadcasted_iota(jnp.int32, (_QB, _K), 1)

    def extract(t, carry):
        cols, kd2 = carry
        mf = m1_ref[...]
        kmin = jax.lax.bitcast_convert_type(jnp.min(mf, axis=1), jnp.int32)
        ml = jnp.argmin(mf, axis=1)
        hit = lane == ml[:, None]
        sel = lane_k == t
        cols = jnp.where(sel, ((kmin & _CMASK) * _L + ml)[:, None], cols)
        kd2 = jnp.where(sel, (kmin & ~_CMASK)[:, None], kd2)
        m1_ref[...] = jnp.where(hit, m2_ref[...], mf)
        m2_ref[...] = jnp.where(hit, big, m2_ref[...])
        return cols, kd2

    cols0 = jnp.zeros((_QB, _K), jnp.int32)
    kd20 = jnp.zeros((_QB, _K), jnp.int32)
    cols, kd2 = jax.lax.fori_loop(0, _K, extract, (cols0, kd20), unroll=False)
    nbr_ref[...] = cols
    w_ref[...] = jnp.exp(-10.0 * jax.lax.bitcast_convert_type(kd2, jnp.float32))


def _run_knn(s, s0t, s1t):
    grid = _NPAD // _QB
    return pl.pallas_call(
        _knn_body,
        grid=(grid,),
        in_specs=[pl.BlockSpec((_QB, 2), lambda i: (i, 0)),
                  pl.BlockSpec(s0t.shape, lambda i: (0, 0)),
                  pl.BlockSpec(s1t.shape, lambda i: (0, 0))],
        out_specs=[pl.BlockSpec((_QB, _K), lambda i: (i, 0)),
                   pl.BlockSpec((_QB, _K), lambda i: (i, 0))],
        out_shape=[jax.ShapeDtypeStruct((_N, _K), jnp.int32),
                   jax.ShapeDtypeStruct((_N, _K), jnp.float32)],
        scratch_shapes=[pltpu.VMEM((_QB, _L), jnp.float32),
                        pltpu.VMEM((_QB, _L), jnp.float32)],
    )(s, s0t, s1t)


def _sc_gather(h, idx_flat):
    n_idx = idx_flat.shape[1]
    dim = h.shape[1]
    mesh = plsc.VectorSubcoreMesh(core_axis_name="core",
                                  subcore_axis_name="subcore")

    @pl.kernel(out_type=jax.ShapeDtypeStruct((n_idx, dim), h.dtype), mesh=mesh)
    def k(h_hbm, i_hbm, o_hbm):
        def body(i_vmem, o_vmem):
            pltpu.sync_copy(h_hbm.at[i_vmem.at[0]], o_vmem)

        pltpu.emit_pipeline(
            body,
            grid=(n_idx // _GW,),
            in_specs=[pl.BlockSpec((1, _GW), lambda i: (0, i))],
            out_specs=[pl.BlockSpec((_GW, dim), lambda i: (i, 0))],
            core_axis_name=("core", "subcore"),
            dimension_semantics=(pltpu.PARALLEL,),
        )(i_hbm, o_hbm)

    return k(h, idx_flat)


def _heads_body(g_ref, w_ref, h1_ref, xs_ref,
                wo1, wo2a, wo2b, bo2,
                n2w1, n2b1, n2w2, n2b2, n2w3, n2b3, n2w4, n2b4,
                n3w1h, n3w1i, n3b1, n3w2, n3b2, n3w3, n3b3, n3w4, n3b4,
                ids_ref, p4_ref):
    msg0 = g_ref[:, 0, 0:32] * w_ref[:, 0:1]
    acc = msg0
    mx = msg0
    for j in range(1, _K):
        msg = g_ref[:, j, 0:32] * w_ref[:, j:j + 1]
        acc = acc + msg
        mx = jnp.maximum(mx, msg)
    mean = acc * (1.0 / _K)
    h2 = (jnp.dot(h1_ref[...], wo1[...], preferred_element_type=jnp.float32)
          + jnp.dot(mean, wo2a[...], preferred_element_type=jnp.float32)
          + jnp.dot(mx, wo2b[...], preferred_element_type=jnp.float32)
          + bo2[...])
    h2 = _leaky(h2)
    t = _leaky(jnp.dot(h2, n2w1[...], preferred_element_type=jnp.float32) + n2b1[...])
    t = _leaky(jnp.dot(t, n2w2[...], preferred_element_type=jnp.float32) + n2b2[...])
    t = _leaky(jnp.dot(t, n2w3[...], preferred_element_type=jnp.float32) + n2b3[...])
    ids = jnp.dot(t, n2w4[...], preferred_element_type=jnp.float32) + n2b4[...]
    ids_ref[...] = ids
    u = _leaky(jnp.dot(h2, n3w1h[...], preferred_element_type=jnp.float32)
               + jnp.dot(ids, n3w1i[...], preferred_element_type=jnp.float32)
               + n3b1[...])
    u = _leaky(jnp.dot(u, n3w2[...], preferred_element_type=jnp.float32) + n3b2[...])
    u = _leaky(jnp.dot(u, n3w3[...], preferred_element_type=jnp.float32) + n3b3[...])
    p4_ref[...] = (jnp.dot(u, n3w4[...], preferred_element_type=jnp.float32)
                   + n3b4[...] + xs_ref[...])


def _run_heads(g3, w, h1, xs, conv, nn2, nn3):
    n = h1.shape[0]
    grid = n // _RB
    Ws, bs, Wh, bh, Wo1, Wo2, bo2 = conv
    full = lambda a: pl.BlockSpec(a.shape, lambda i: (0,) * a.ndim)
    row = lambda d: pl.BlockSpec((_RB, d), lambda i: (i, 0))
    wargs = [Wo1.T, Wo2.T[:32, :], Wo2.T[32:, :], bo2.reshape(1, -1)]
    for (W, b) in nn2:
        wargs += [W.T, b.reshape(1, -1)]
    (V1, c1), (V2, c2), (V3, c3), (V4, c4) = nn3
    wargs += [V1.T[:32, :], V1.T[32:, :], c1.reshape(1, -1),
              V2.T, c2.reshape(1, -1), V3.T, c3.reshape(1, -1),
              V4.T, c4.reshape(1, -1)]
    in_specs = [pl.BlockSpec((_RB, _K, 128), lambda i: (i, 0, 0)),
                row(_K), row(32), row(4)]
    in_specs += [full(a) for a in wargs]
    return pl.pallas_call(
        _heads_body,
        grid=(grid,),
        in_specs=in_specs,
        out_specs=[row(6), row(4)],
        out_shape=[jax.ShapeDtypeStruct((n, 6), jnp.float32),
                   jax.ShapeDtypeStruct((n, 4), jnp.float32)],
    )(g3, w, h1, xs, *wargs)


def kernel(x, nn1, conv, nn2, nn3):
    Ws, bs, Wh, bh, Wo1, Wo2, bo2 = conv
    h1, h, s = _run_mlp_proj(x, nn1, Ws, bs, Wh, bh)
    pad = jnp.full((_NPAD - _N,), 1e18, dtype=jnp.float32)
    s0t = jnp.concatenate([s[:, 0], pad]).reshape(_FC, _L)
    s1t = jnp.concatenate([s[:, 1], pad]).reshape(_FC, _L)
    nbr, w = _run_knn(s, s0t, s1t)
    g = _sc_gather(h, nbr.reshape(1, _N * _K).astype(jnp.int32))
    g3 = g.reshape(_N, _K, 128)
    ids, p4 = _run_heads(g3, w, h1, x[:, 3:7], conv, nn2, nn3)
    return (ids, p4)


# speedup vs baseline: 19.1720x; 1.0706x over previous
import jax
import jax.numpy as jnp
from jax.experimental import pallas as pl
from jax.experimental.pallas import tpu as pltpu
from jax.experimental.pallas import tpu_sc as plsc

_NEG = 0.01
_N = 10000
_K = 16
_QB = 5120
_NPAD = 10240
_RB = 1000
_GW = 128


def _leaky(v):
    return jnp.where(v >= 0, v, _NEG * v)


def _mlp_proj_body(x_ref, w1, b1, w2, b2, w3, b3, w4, b4, wst, bs, wht, bh,
                   h1_ref, h_ref, s_ref):
    h = _leaky(jnp.dot(x_ref[...], w1[...],
                       preferred_element_type=jnp.float32) + b1[...])
    h = _leaky(jnp.dot(h, w2[...], preferred_element_type=jnp.float32) + b2[...])
    h = _leaky(jnp.dot(h, w3[...], preferred_element_type=jnp.float32) + b3[...])
    h = _leaky(jnp.dot(h, w4[...], preferred_element_type=jnp.float32) + b4[...])
    h1_ref[...] = h
    s_ref[...] = jnp.dot(h, wst[...], preferred_element_type=jnp.float32) + bs[...]
    h_ref[...] = jnp.dot(h, wht[...], preferred_element_type=jnp.float32) + bh[...]


def _run_mlp_proj(x, nn1, Ws, bs, Wh, bh):
    n = x.shape[0]
    grid = n // _RB
    full = lambda a: pl.BlockSpec(a.shape, lambda i: (0,) * a.ndim)
    row = lambda d: pl.BlockSpec((_RB, d), lambda i: (i, 0))
    wargs = []
    in_specs = [row(x.shape[1])]
    for (W, b) in nn1:
        wargs += [W.T, b.reshape(1, -1)]
    wht = jnp.zeros((32, 128), jnp.float32).at[:, :32].set(Wh.T)
    bht = jnp.zeros((1, 128), jnp.float32).at[:, :32].set(bh.reshape(1, -1))
    wargs += [Ws.T, bs.reshape(1, -1), wht, bht]
    in_specs += [full(a) for a in wargs]
    return pl.pallas_call(
        _mlp_proj_body,
        grid=(grid,),
        in_specs=in_specs,
        out_specs=[row(32), row(128), row(2)],
        out_shape=[jax.ShapeDtypeStruct((n, 32), jnp.float32),
                   jax.ShapeDtypeStruct((n, 128), jnp.float32),
                   jax.ShapeDtypeStruct((n, 2), jnp.float32)],
    )(x, *wargs)


_L = 256
_FC = _NPAD // _L
_CMASK = 63


def _knn_body(sq_ref, s0t_ref, s1t_ref, nbr_ref, w_ref, m1_ref, m2_ref):
    s0q = sq_ref[:, 0:1]
    s1q = sq_ref[:, 1:2]
    big = jnp.full((_QB, _L), 3.0e38, jnp.float32)
    m1_ref[...] = big
    m2_ref[...] = big

    def key_for(c):
        s0a = s0t_ref[pl.ds(c, 1), :]
        s1a = s1t_ref[pl.ds(c, 1), :]
        d0 = s0q - s0a
        d1 = s1q - s1a
        d2c = d0 * d0 + d1 * d1
        kb = jax.lax.bitcast_convert_type(d2c, jnp.int32)
        return jax.lax.bitcast_convert_type((kb & ~_CMASK) | c, jnp.float32)

    def fold4(i, _):
        m1 = m1_ref[...]
        m2 = m2_ref[...]
        for dc in range(4):
            k = key_for(i * 4 + dc)
            m2 = jnp.minimum(m2, jnp.maximum(m1, k))
            m1 = jnp.minimum(m1, k)
        m1_ref[...] = m1
        m2_ref[...] = m2
        return 0

    jax.lax.fori_loop(0, _FC // 4, fold4, 0, unroll=False)

    lane = jax.lax.broadcasted_iota(jnp.int32, (_QB, _L), 1)
    lane_k = jax.lax.broadcasted_iota(jnp.int32, (_QB, _K), 1)

    def extract(t, carry):
        cols, kd2 = carry
        mf = m1_ref[...]
        kmin = jax.lax.bitcast_convert_type(jnp.min(mf, axis=1), jnp.int32)
        ml = jnp.argmin(mf, axis=1)
        hit = lane == ml[:, None]
        sel = lane_k == t
        cols = jnp.where(sel, ((kmin & _CMASK) * _L + ml)[:, None], cols)
        kd2 = jnp.where(sel, (kmin & ~_CMASK)[:, None], kd2)
        m1_ref[...] = jnp.where(hit, m2_ref[...], mf)
        m2_ref[...] = jnp.where(hit, big, m2_ref[...])
        return cols, kd2

    cols0 = jnp.zeros((_QB, _K), jnp.int32)
    kd20 = jnp.zeros((_QB, _K), jnp.int32)
    cols, kd2 = jax.lax.fori_loop(0, _K, extract, (cols0, kd20), unroll=False)
    nbr_ref[...] = cols
    w_ref[...] = jnp.exp(-10.0 * jax.lax.bitcast_convert_type(kd2, jnp.float32))


def _run_knn(s, s0t, s1t):
    grid = _NPAD // _QB
    return pl.pallas_call(
        _knn_body,
        grid=(grid,),
        in_specs=[pl.BlockSpec((_QB, 2), lambda i: (i, 0)),
                  pl.BlockSpec(s0t.shape, lambda i: (0, 0)),
                  pl.BlockSpec(s1t.shape, lambda i: (0, 0))],
        out_specs=[pl.BlockSpec((_QB, _K), lambda i: (i, 0)),
                   pl.BlockSpec((_QB, _K), lambda i: (i, 0))],
        out_shape=[jax.ShapeDtypeStruct((_N, _K), jnp.int32),
                   jax.ShapeDtypeStruct((_N, _K), jnp.float32)],
        scratch_shapes=[pltpu.VMEM((_QB, _L), jnp.float32),
                        pltpu.VMEM((_QB, _L), jnp.float32)],
    )(s, s0t, s1t)


def _sc_gather(h, idx_flat):
    n_idx = idx_flat.shape[1]
    dim = h.shape[1]
    mesh = plsc.VectorSubcoreMesh(core_axis_name="core",
                                  subcore_axis_name="subcore")

    @pl.kernel(out_type=jax.ShapeDtypeStruct((n_idx, dim), h.dtype), mesh=mesh)
    def k(h_hbm, i_hbm, o_hbm):
        def body(i_vmem, o_vmem):
            pltpu.sync_copy(h_hbm.at[i_vmem.at[0]], o_vmem)

        pltpu.emit_pipeline(
            body,
            grid=(n_idx // _GW,),
            in_specs=[pl.BlockSpec((1, _GW), lambda i: (0, i))],
            out_specs=[pl.BlockSpec((_GW, dim), lambda i: (i, 0))],
            core_axis_name=("core", "subcore"),
            dimension_semantics=(pltpu.PARALLEL,),
        )(i_hbm, o_hbm)

    return k(h, idx_flat)


def _heads_body(g_ref, w_ref, h1_ref, xs_ref,
                wo1, wo2a, wo2b, bo2,
                n2w1, n2b1, n2w2, n2b2, n2w3, n2b3, n2w4, n2b4,
                n3w1h, n3w1i, n3b1, n3w2, n3b2, n3w3, n3b3, n3w4, n3b4,
                ids_ref, p4_ref):
    msg0 = g_ref[:, 0, 0:32] * w_ref[:, 0:1]
    acc = msg0
    mx = msg0
    for j in range(1, _K):
        msg = g_ref[:, j, 0:32] * w_ref[:, j:j + 1]
        acc = acc + msg
        mx = jnp.maximum(mx, msg)
    mean = acc * (1.0 / _K)
    h2 = (jnp.dot(h1_ref[...], wo1[...], preferred_element_type=jnp.float32)
          + jnp.dot(mean, wo2a[...], preferred_element_type=jnp.float32)
          + jnp.dot(mx, wo2b[...], preferred_element_type=jnp.float32)
          + bo2[...])
    h2 = _leaky(h2)
    t = _leaky(jnp.dot(h2, n2w1[...], preferred_element_type=jnp.float32) + n2b1[...])
    t = _leaky(jnp.dot(t, n2w2[...], preferred_element_type=jnp.float32) + n2b2[...])
    t = _leaky(jnp.dot(t, n2w3[...], preferred_element_type=jnp.float32) + n2b3[...])
    ids = jnp.dot(t, n2w4[...], preferred_element_type=jnp.float32) + n2b4[...]
    ids_ref[...] = ids
    u = _leaky(jnp.dot(h2, n3w1h[...], preferred_element_type=jnp.float32)
               + jnp.dot(ids, n3w1i[...], preferred_element_type=jnp.float32)
               + n3b1[...])
    u = _leaky(jnp.dot(u, n3w2[...], preferred_element_type=jnp.float32) + n3b2[...])
    u = _leaky(jnp.dot(u, n3w3[...], preferred_element_type=jnp.float32) + n3b3[...])
    p4_ref[...] = (jnp.dot(u, n3w4[...], preferred_element_type=jnp.float32)
                   + n3b4[...] + xs_ref[...])


def _run_heads(g3, w, h1, xs, conv, nn2, nn3):
    n = h1.shape[0]
    grid = n // _RB
    Ws, bs, Wh, bh, Wo1, Wo2, bo2 = conv
    full = lambda a: pl.BlockSpec(a.shape, lambda i: (0,) * a.ndim)
    row = lambda d: pl.BlockSpec((_RB, d), lambda i: (i, 0))
    wargs = [Wo1.T, Wo2.T[:32, :], Wo2.T[32:, :], bo2.reshape(1, -1)]
    for (W, b) in nn2:
        wargs += [W.T, b.reshape(1, -1)]
    (V1, c1), (V2, c2), (V3, c3), (V4, c4) = nn3
    wargs += [V1.T[:32, :], V1.T[32:, :], c1.reshape(1, -1),
              V2.T, c2.reshape(1, -1), V3.T, c3.reshape(1, -1),
              V4.T, c4.reshape(1, -1)]
    in_specs = [pl.BlockSpec((_RB, _K, 128), lambda i: (i, 0, 0)),
                row(_K), row(32), row(4)]
    in_specs += [full(a) for a in wargs]
    return pl.pallas_call(
        _heads_body,
        grid=(grid,),
        in_specs=in_specs,
        out_specs=[row(6), row(4)],
        out_shape=[jax.ShapeDtypeStruct((n, 6), jnp.float32),
                   jax.ShapeDtypeStruct((n, 4), jnp.float32)],
    )(g3, w, h1, xs, *wargs)


def kernel(x, nn1, conv, nn2, nn3):
    Ws, bs, Wh, bh, Wo1, Wo2, bo2 = conv
    h1, h, s = _run_mlp_proj(x, nn1, Ws, bs, Wh, bh)
    pad = jnp.full((_NPAD - _N,), 1e18, dtype=jnp.float32)
    s0t = jnp.concatenate([s[:, 0], pad]).reshape(_FC, _L)
    s1t = jnp.concatenate([s[:, 1], pad]).reshape(_FC, _L)
    nbr, w = _run_knn(s, s0t, s1t)
    g = _sc_gather(h, nbr.reshape(1, _N * _K).astype(jnp.int32))
    g3 = g.reshape(_N, _K, 128)
    ids, p4 = _run_heads(g3, w, h1, x[:, 3:7], conv, nn2, nn3)
    return (ids, p4)


# speedup vs baseline: 20.1588x; 1.0515x over previous
import jax
import jax.numpy as jnp
from jax.experimental import pallas as pl
from jax.experimental.pallas import tpu as pltpu
from jax.experimental.pallas import tpu_sc as plsc

_NEG = 0.01
_N = 10000
_K = 16
_QB = 5120
_NPAD = 10240
_RB = 1000
_GW = 128


def _leaky(v):
    return jnp.where(v >= 0, v, _NEG * v)


def _mlp_proj_body(x_ref, w1, b1, w2, b2, w3, b3, w4, b4, wst, bs, wht, bh,
                   h1_ref, h_ref, s_ref):
    h = _leaky(jnp.dot(x_ref[...], w1[...],
                       preferred_element_type=jnp.float32) + b1[...])
    h = _leaky(jnp.dot(h, w2[...], preferred_element_type=jnp.float32) + b2[...])
    h = _leaky(jnp.dot(h, w3[...], preferred_element_type=jnp.float32) + b3[...])
    h = _leaky(jnp.dot(h, w4[...], preferred_element_type=jnp.float32) + b4[...])
    h1_ref[...] = h
    s_ref[...] = jnp.dot(h, wst[...], preferred_element_type=jnp.float32) + bs[...]
    h_ref[...] = jnp.dot(h, wht[...], preferred_element_type=jnp.float32) + bh[...]


def _run_mlp_proj(x, nn1, Ws, bs, Wh, bh):
    n = x.shape[0]
    grid = n // _RB
    full = lambda a: pl.BlockSpec(a.shape, lambda i: (0,) * a.ndim)
    row = lambda d: pl.BlockSpec((_RB, d), lambda i: (i, 0))
    wargs = []
    in_specs = [row(x.shape[1])]
    for (W, b) in nn1:
        wargs += [W.T, b.reshape(1, -1)]
    wht = jnp.zeros((32, 128), jnp.float32).at[:, :32].set(Wh.T)
    bht = jnp.zeros((1, 128), jnp.float32).at[:, :32].set(bh.reshape(1, -1))
    wargs += [Ws.T, bs.reshape(1, -1), wht, bht]
    in_specs += [full(a) for a in wargs]
    return pl.pallas_call(
        _mlp_proj_body,
        grid=(grid,),
        in_specs=in_specs,
        out_specs=[row(32), row(128), row(2)],
        out_shape=[jax.ShapeDtypeStruct((n, 32), jnp.float32),
                   jax.ShapeDtypeStruct((n, 128), jnp.float32),
                   jax.ShapeDtypeStruct((n, 2), jnp.float32)],
    )(x, *wargs)


_L = 256
_FC = _NPAD // _L
_CMASK = 63


def _knn_body(sq_ref, s0t_ref, s1t_ref, nbr_ref, w_ref, m1_ref, m2_ref):
    s0q = sq_ref[:, 0:1]
    s1q = sq_ref[:, 1:2]
    big = jnp.full((_QB, _L), 3.0e38, jnp.float32)
    m1_ref[...] = big
    m2_ref[...] = big

    def key_for(c):
        s0a = s0t_ref[pl.ds(c, 1), :]
        s1a = s1t_ref[pl.ds(c, 1), :]
        d0 = s0q - s0a
        d1 = s1q - s1a
        d2c = d0 * d0 + d1 * d1
        kb = jax.lax.bitcast_convert_type(d2c, jnp.int32)
        return jax.lax.bitcast_convert_type((kb & ~_CMASK) | c, jnp.float32)

    def fold4(i, _):
        m1 = m1_ref[...]
        m2 = m2_ref[...]
        for dc in range(4):
            k = key_for(i * 4 + dc)
            m2 = jnp.minimum(m2, jnp.maximum(m1, k))
            m1 = jnp.minimum(m1, k)
        m1_ref[...] = m1
        m2_ref[...] = m2
        return 0

    jax.lax.fori_loop(0, _FC // 4, fold4, 0, unroll=False)

    lane = jax.lax.broadcasted_iota(jnp.int32, (_QB, _L), 1)
    lane_k = jax.lax.broadcasted_iota(jnp.int32, (_QB, _K), 1)

    def extract(t, carry):
        cols, kd2 = carry
        mf = m1_ref[...]
        kmin = jax.lax.bitcast_convert_type(jnp.min(mf, axis=1), jnp.int32)
        ml = jnp.argmin(mf, axis=1)
        hit = lane == ml[:, None]
        sel = lane_k == t
        cols = jnp.where(sel, ((kmin & _CMASK) * _L + ml)[:, None], cols)
        kd2 = jnp.where(sel, (kmin & ~_CMASK)[:, None], kd2)
        m1_ref[...] = jnp.where(hit, m2_ref[...], mf)
        m2_ref[...] = jnp.where(hit, big, m2_ref[...])
        return cols, kd2

    cols0 = jnp.zeros((_QB, _K), jnp.int32)
    kd20 = jnp.zeros((_QB, _K), jnp.int32)
    cols, kd2 = jax.lax.fori_loop(0, _K, extract, (cols0, kd20), unroll=False)
    nbr_ref[...] = cols
    w_ref[...] = jnp.exp(-10.0 * jax.lax.bitcast_convert_type(kd2, jnp.float32))


def _run_knn(s_q, s0t, s1t):
    n_q = s_q.shape[0]
    return pl.pallas_call(
        _knn_body,
        grid=(1,),
        in_specs=[pl.BlockSpec((_QB, 2), lambda i: (0, 0)),
                  pl.BlockSpec(s0t.shape, lambda i: (0, 0)),
                  pl.BlockSpec(s1t.shape, lambda i: (0, 0))],
        out_specs=[pl.BlockSpec((_QB, _K), lambda i: (0, 0)),
                   pl.BlockSpec((_QB, _K), lambda i: (0, 0))],
        out_shape=[jax.ShapeDtypeStruct((n_q, _K), jnp.int32),
                   jax.ShapeDtypeStruct((n_q, _K), jnp.float32)],
        scratch_shapes=[pltpu.VMEM((_QB, _L), jnp.float32),
                        pltpu.VMEM((_QB, _L), jnp.float32)],
    )(s_q, s0t, s1t)


def _sc_gather(h, idx_flat):
    n_idx = idx_flat.shape[1]
    dim = h.shape[1]
    mesh = plsc.VectorSubcoreMesh(core_axis_name="core",
                                  subcore_axis_name="subcore")

    @pl.kernel(out_type=jax.ShapeDtypeStruct((n_idx, dim), h.dtype), mesh=mesh)
    def k(h_hbm, i_hbm, o_hbm):
        def body(i_vmem, o_vmem):
            pltpu.sync_copy(h_hbm.at[i_vmem.at[0]], o_vmem)

        pltpu.emit_pipeline(
            body,
            grid=(n_idx // _GW,),
            in_specs=[pl.BlockSpec((1, _GW), lambda i: (0, i))],
            out_specs=[pl.BlockSpec((_GW, dim), lambda i: (i, 0))],
            core_axis_name=("core", "subcore"),
            dimension_semantics=(pltpu.PARALLEL,),
        )(i_hbm, o_hbm)

    return k(h, idx_flat)


def _heads_body(g_ref, w_ref, h1_ref, xs_ref,
                wo1, wo2a, wo2b, bo2,
                n2w1, n2b1, n2w2, n2b2, n2w3, n2b3, n2w4, n2b4,
                n3w1h, n3w1i, n3b1, n3w2, n3b2, n3w3, n3b3, n3w4, n3b4,
                ids_ref, p4_ref):
    msg0 = g_ref[:, 0, 0:32] * w_ref[:, 0:1]
    acc = msg0
    mx = msg0
    for j in range(1, _K):
        msg = g_ref[:, j, 0:32] * w_ref[:, j:j + 1]
        acc = acc + msg
        mx = jnp.maximum(mx, msg)
    mean = acc * (1.0 / _K)
    h2 = (jnp.dot(h1_ref[...], wo1[...], preferred_element_type=jnp.float32)
          + jnp.dot(mean, wo2a[...], preferred_element_type=jnp.float32)
          + jnp.dot(mx, wo2b[...], preferred_element_type=jnp.float32)
          + bo2[...])
    h2 = _leaky(h2)
    t = _leaky(jnp.dot(h2, n2w1[...], preferred_element_type=jnp.float32) + n2b1[...])
    t = _leaky(jnp.dot(t, n2w2[...], preferred_element_type=jnp.float32) + n2b2[...])
    t = _leaky(jnp.dot(t, n2w3[...], preferred_element_type=jnp.float32) + n2b3[...])
    ids = jnp.dot(t, n2w4[...], preferred_element_type=jnp.float32) + n2b4[...]
    ids_ref[...] = ids
    u = _leaky(jnp.dot(h2, n3w1h[...], preferred_element_type=jnp.float32)
               + jnp.dot(ids, n3w1i[...], preferred_element_type=jnp.float32)
               + n3b1[...])
    u = _leaky(jnp.dot(u, n3w2[...], preferred_element_type=jnp.float32) + n3b2[...])
    u = _leaky(jnp.dot(u, n3w3[...], preferred_element_type=jnp.float32) + n3b3[...])
    p4_ref[...] = (jnp.dot(u, n3w4[...], preferred_element_type=jnp.float32)
                   + n3b4[...] + xs_ref[...])


def _run_heads(g3, w, h1, xs, conv, nn2, nn3):
    n = h1.shape[0]
    grid = n // _RB
    Ws, bs, Wh, bh, Wo1, Wo2, bo2 = conv
    full = lambda a: pl.BlockSpec(a.shape, lambda i: (0,) * a.ndim)
    row = lambda d: pl.BlockSpec((_RB, d), lambda i: (i, 0))
    wargs = [Wo1.T, Wo2.T[:32, :], Wo2.T[32:, :], bo2.reshape(1, -1)]
    for (W, b) in nn2:
        wargs += [W.T, b.reshape(1, -1)]
    (V1, c1), (V2, c2), (V3, c3), (V4, c4) = nn3
    wargs += [V1.T[:32, :], V1.T[32:, :], c1.reshape(1, -1),
              V2.T, c2.reshape(1, -1), V3.T, c3.reshape(1, -1),
              V4.T, c4.reshape(1, -1)]
    in_specs = [pl.BlockSpec((_RB, _K, 128), lambda i: (i, 0, 0)),
                row(_K), row(32), row(4)]
    in_specs += [full(a) for a in wargs]
    return pl.pallas_call(
        _heads_body,
        grid=(grid,),
        in_specs=in_specs,
        out_specs=[row(6), row(4)],
        out_shape=[jax.ShapeDtypeStruct((n, 6), jnp.float32),
                   jax.ShapeDtypeStruct((n, 4), jnp.float32)],
    )(g3, w, h1, xs, *wargs)


def kernel(x, nn1, conv, nn2, nn3):
    Ws, bs, Wh, bh, Wo1, Wo2, bo2 = conv
    h1, h, s = _run_mlp_proj(x, nn1, Ws, bs, Wh, bh)
    pad = jnp.full((_NPAD - _N,), 1e18, dtype=jnp.float32)
    s0t = jnp.concatenate([s[:, 0], pad]).reshape(_FC, _L)
    s1t = jnp.concatenate([s[:, 1], pad]).reshape(_FC, _L)
    half = _N // 2
    xs = x[:, 3:7]
    nbr0, w0 = _run_knn(s[:half], s0t, s1t)
    nbr1, w1 = _run_knn(s[half:], s0t, s1t)
    g0 = _sc_gather(h, nbr0.reshape(1, half * _K))
    g1 = _sc_gather(h, nbr1.reshape(1, half * _K))
    ids0, p40 = _run_heads(g0.reshape(half, _K, 128), w0, h1[:half],
                           xs[:half], conv, nn2, nn3)
    ids1, p41 = _run_heads(g1.reshape(half, _K, 128), w1, h1[half:],
                           xs[half:], conv, nn2, nn3)
    return (jnp.concatenate([ids0, ids1], axis=0),
            jnp.concatenate([p40, p41], axis=0))
